# baseline (speedup 1.0000x reference)
import functools

import jax
import jax.numpy as jnp
from jax import lax
from jax.experimental import pallas as pl
from jax.experimental.pallas import tpu as pltpu
from jax.experimental.pallas import tpu_sc as plsc

EMBED_DIM = 64
HALF = 32
NC = 2
NS = 16
NW = NC * NS
BLK = 128
CH = 64
TCOL = 128


def _make_repack(vocab: int):
    n_full = vocab // TCOL
    per_w = (n_full + NW - 1) // NW
    mesh = plsc.VectorSubcoreMesh(core_axis_name="c", subcore_axis_name="s")

    @functools.partial(
        pl.kernel,
        mesh=mesh,
        out_type=jax.ShapeDtypeStruct((vocab // 2, 128), jnp.float32),
        scratch_types=[
            pltpu.VMEM((EMBED_DIM, TCOL), jnp.float32),
            pltpu.VMEM((TCOL // 2, 128), jnp.float32),
        ],
        compiler_params=pltpu.CompilerParams(
            use_tc_tiling_on_sc=True, needs_layout_passes=False
        ),
    )
    def repack_kernel(tt_hbm, tail_hbm, out_hbm, src_v, dst_v):
        wid = lax.axis_index("s") * NC + lax.axis_index("c")
        lanes = lax.iota(jnp.int32, 16)

        def do_chunk(col0, out0):
            pltpu.sync_copy(tt_hbm.at[:, pl.ds(col0, TCOL)], src_v)

            def row(r, carry):
                for g in range(8):
                    e = g // 4
                    x = plsc.load_gather(
                        src_v, [lanes + 16 * (g % 4), lanes * 0 + (2 * r + e)]
                    )
                    dst_v[r, pl.ds(16 * g, 16)] = x
                return carry

            lax.fori_loop(0, TCOL // 2, row, 0)
            pltpu.sync_copy(dst_v, out_hbm.at[pl.ds(out0, TCOL // 2)])

        def body(i, carry):
            c = i * NW + wid

            @pl.when(c < n_full)
            def _():
                do_chunk(c * TCOL, c * (TCOL // 2))

            return carry

        lax.fori_loop(0, per_w, body, 0)

        n_tail = (vocab - n_full * TCOL) // 2
        if n_tail:
            @pl.when(wid == NW - 1)
            def _():
                pltpu.sync_copy(tail_hbm, dst_v.at[pl.ds(0, n_tail)])
                pltpu.sync_copy(
                    dst_v.at[pl.ds(0, n_tail)],
                    out_hbm.at[pl.ds(n_full * (TCOL // 2), n_tail)],
                )

    return repack_kernel


def _make_gather(n_batch: int, n_seq: int, vocab: int):
    mesh = plsc.VectorSubcoreMesh(core_axis_name="c", subcore_axis_name="s")

    @functools.partial(
        pl.kernel,
        mesh=mesh,
        out_type=jax.ShapeDtypeStruct((n_batch * n_seq * 2, HALF), jnp.float32),
        scratch_types=[
            pltpu.VMEM((n_seq, BLK), jnp.int32),
            pltpu.VMEM((2 * CH,), jnp.int32),
            pltpu.VMEM((2 * CH, HALF), jnp.float32),
            pltpu.SemaphoreType.DMA,
        ],
        compiler_params=pltpu.CompilerParams(
            use_tc_tiling_on_sc=False, needs_layout_passes=False
        ),
    )
    def gather_kernel(tokens_t_hbm, table_hbm, out_hbm, idx_v, idx32_v, rows_v, sem):
        wid = lax.axis_index("s") * NC + lax.axis_index("c")
        base = wid * BLK
        pltpu.sync_copy(tokens_t_hbm.at[:, pl.ds(base, BLK)], idx_v)
        lanes = lax.iota(jnp.int32, 16)

        def body(c, carry):
            j = c // (BLK // CH)
            h = c % (BLK // CH)
            for g in range(CH // 16):
                b = idx_v[j, pl.ds(h * CH + g * 16, 16)] * 2
                plsc.store_scatter(idx32_v, [lanes * 2 + g * 32], b)
                plsc.store_scatter(idx32_v, [lanes * 2 + 1 + g * 32], b + 1)
            pltpu.async_copy(table_hbm.at[idx32_v], rows_v, sem).wait()
            pltpu.sync_copy(
                rows_v,
                out_hbm.at[pl.ds(2 * (j * n_batch + base + h * CH), 2 * CH)],
            )
            return carry

        lax.fori_loop(0, n_seq * (BLK // CH), body, 0)

    return gather_kernel


def kernel(tokens, table):
    n_batch, n_seq = tokens.shape
    vocab = table.shape[0]
    assert n_batch % NW == 0 and n_batch // NW == BLK
    tokens_t = tokens.T.astype(jnp.int32)
    covered = vocab // TCOL * TCOL
    tail128 = table[covered:].reshape(-1, 128)
    table128 = _make_repack(vocab)(table.T, tail128)
    table32 = table128.reshape(2 * vocab, HALF)
    out = _make_gather(n_batch, n_seq, vocab)(tokens_t, table32)
    return out.reshape(n_seq, n_batch, EMBED_DIM).transpose(1, 0, 2)


# speedup vs baseline: 1.1905x; 1.1905x over previous
import functools

import jax
import jax.numpy as jnp
from jax import lax
from jax.experimental import pallas as pl
from jax.experimental.pallas import tpu as pltpu
from jax.experimental.pallas import tpu_sc as plsc

EMBED_DIM = 64
HALF = 32
NC = 2
NS = 16
NW = NC * NS
BLK = 128
CH = 64
TCOL = 256


def _make_repack(vocab: int):
    n_full = vocab // TCOL
    per_w = (n_full + NW - 1) // NW
    per_w += per_w % 2
    mesh = plsc.VectorSubcoreMesh(core_axis_name="c", subcore_axis_name="s")

    @functools.partial(
        pl.kernel,
        mesh=mesh,
        out_type=jax.ShapeDtypeStruct((vocab // 2, 128), jnp.float32),
        scratch_types=[
            pltpu.VMEM((2, EMBED_DIM, TCOL), jnp.float32),
            pltpu.VMEM((2, TCOL // 2, 128), jnp.float32),
            pltpu.SemaphoreType.DMA,
            pltpu.SemaphoreType.DMA,
            pltpu.SemaphoreType.DMA,
            pltpu.SemaphoreType.DMA,
        ],
        compiler_params=pltpu.CompilerParams(
            use_tc_tiling_on_sc=True, needs_layout_passes=False
        ),
    )
    def repack_kernel(tt_hbm, tail_hbm, out_hbm, src_v, dst_v, si0, si1, so0, so1):
        wid = lax.axis_index("s") * NC + lax.axis_index("c")
        lanes = lax.iota(jnp.int32, 16)
        dvecs = [lanes + 16 * k for k in range(4)]
        sin = (si0, si1)
        sout = (so0, so1)

        def chunk_of(i):
            return lax.rem(i * NW + wid, n_full)

        def in_copy(i, b):
            c = chunk_of(i)
            return pltpu.make_async_copy(
                tt_hbm.at[:, pl.ds(c * TCOL, TCOL)], src_v.at[b], sin[b]
            )

        def out_copy(i, b):
            c = chunk_of(i)
            return pltpu.make_async_copy(
                dst_v.at[b], out_hbm.at[pl.ds(c * (TCOL // 2), TCOL // 2)], sout[b]
            )

        def transpose(b):
            src = src_v.at[b]
            dst = dst_v.at[b]

            def row(r2, carry):
                for u in range(2):
                    r = 2 * r2 + u
                    for g in range(8):
                        e = g // 4
                        x = plsc.load_gather(
                            src, [dvecs[g % 4], lanes * 0 + (2 * r + e)]
                        )
                        dst[r, pl.ds(16 * g, 16)] = x
                return carry

            lax.fori_loop(0, TCOL // 4, row, 0)

        in_copy(0, 0).start()

        def body(i2, carry):
            for b in range(2):
                i = 2 * i2 + b

                @pl.when(i + 1 < per_w)
                def _():
                    in_copy(i + 1, 1 - b).start()

                in_copy(i, b).wait()

                @pl.when(i >= 2)
                def _():
                    out_copy(i - 2, b).wait()

                transpose(b)
                out_copy(i, b).start()
            return carry

        lax.fori_loop(0, per_w // 2, body, 0)
        out_copy(per_w - 2, 0).wait()
        out_copy(per_w - 1, 1).wait()

        n_tail = (vocab - n_full * TCOL) // 2
        if n_tail:
            @pl.when(wid == NW - 1)
            def _():
                pltpu.sync_copy(tail_hbm, dst_v.at[0].at[pl.ds(0, n_tail)])
                pltpu.sync_copy(
                    dst_v.at[0].at[pl.ds(0, n_tail)],
                    out_hbm.at[pl.ds(n_full * (TCOL // 2), n_tail)],
                )

    return repack_kernel


def _make_gather(n_batch: int, n_seq: int, vocab: int):
    mesh = plsc.VectorSubcoreMesh(core_axis_name="c", subcore_axis_name="s")

    @functools.partial(
        pl.kernel,
        mesh=mesh,
        out_type=jax.ShapeDtypeStruct((n_batch * n_seq * 2, HALF), jnp.float32),
        scratch_types=[
            pltpu.VMEM((n_seq, BLK), jnp.int32),
            pltpu.VMEM((2 * CH,), jnp.int32),
            pltpu.VMEM((2 * CH, HALF), jnp.float32),
            pltpu.SemaphoreType.DMA,
        ],
        compiler_params=pltpu.CompilerParams(
            use_tc_tiling_on_sc=False, needs_layout_passes=False
        ),
    )
    def gather_kernel(tokens_t_hbm, table_hbm, out_hbm, idx_v, idx32_v, rows_v, sem):
        wid = lax.axis_index("s") * NC + lax.axis_index("c")
        base = wid * BLK
        pltpu.sync_copy(tokens_t_hbm.at[:, pl.ds(base, BLK)], idx_v)
        lanes = lax.iota(jnp.int32, 16)

        def body(c, carry):
            j = c // (BLK // CH)
            h = c % (BLK // CH)
            for g in range(CH // 16):
                b = idx_v[j, pl.ds(h * CH + g * 16, 16)] * 2
                plsc.store_scatter(idx32_v, [lanes * 2 + g * 32], b)
                plsc.store_scatter(idx32_v, [lanes * 2 + 1 + g * 32], b + 1)
            pltpu.async_copy(table_hbm.at[idx32_v], rows_v, sem).wait()
            pltpu.sync_copy(
                rows_v,
                out_hbm.at[pl.ds(2 * (j * n_batch + base + h * CH), 2 * CH)],
            )
            return carry

        lax.fori_loop(0, n_seq * (BLK // CH), body, 0)

    return gather_kernel


def kernel(tokens, table):
    n_batch, n_seq = tokens.shape
    vocab = table.shape[0]
    assert n_batch % NW == 0 and n_batch // NW == BLK
    tokens_t = tokens.T.astype(jnp.int32)
    covered = vocab // TCOL * TCOL
    tail128 = table[covered:].reshape(-1, 128)
    table128 = _make_repack(vocab)(table.T, tail128)
    table32 = table128.reshape(2 * vocab, HALF)
    out = _make_gather(n_batch, n_seq, vocab)(tokens_t, table32)
    return out.reshape(n_seq, n_batch, EMBED_DIM).transpose(1, 0, 2)


# speedup vs baseline: 4.9228x; 4.1351x over previous
import functools

import jax
import jax.numpy as jnp
from jax import lax
from jax.experimental import pallas as pl
from jax.experimental.pallas import tpu as pltpu
from jax.experimental.pallas import tpu_sc as plsc

EMBED_DIM = 64
HALF = 32
NC = 2
NS = 16
NW = NC * NS
BLK = 128
CH = 64
TCOL = 256


def _make_repack(vocab: int):
    n_full = vocab // TCOL
    per_w = (n_full + NW - 1) // NW
    per_w += per_w % 2
    mesh = plsc.VectorSubcoreMesh(core_axis_name="c", subcore_axis_name="s")

    @functools.partial(
        pl.kernel,
        mesh=mesh,
        out_type=jax.ShapeDtypeStruct((vocab // 2, 128), jnp.float32),
        scratch_types=[
            pltpu.VMEM((2, EMBED_DIM, TCOL), jnp.float32),
            pltpu.VMEM((2, TCOL // 2, 128), jnp.float32),
            pltpu.SemaphoreType.DMA,
            pltpu.SemaphoreType.DMA,
            pltpu.SemaphoreType.DMA,
            pltpu.SemaphoreType.DMA,
        ],
        compiler_params=pltpu.CompilerParams(
            use_tc_tiling_on_sc=True, needs_layout_passes=False
        ),
    )
    def repack_kernel(tt_hbm, tail_hbm, out_hbm, src_v, dst_v, si0, si1, so0, so1):
        wid = lax.axis_index("s") * NC + lax.axis_index("c")
        lanes = lax.iota(jnp.int32, 16)
        dvecs = [lanes + 16 * k for k in range(4)]
        sin = (si0, si1)
        sout = (so0, so1)

        def chunk_of(i):
            return lax.rem(i * NW + wid, n_full)

        def in_copy(i, b):
            c = chunk_of(i)
            return pltpu.make_async_copy(
                tt_hbm.at[:, pl.ds(c * TCOL, TCOL)], src_v.at[b], sin[b]
            )

        def out_copy(i, b):
            c = chunk_of(i)
            return pltpu.make_async_copy(
                dst_v.at[b], out_hbm.at[pl.ds(c * (TCOL // 2), TCOL // 2)], sout[b]
            )

        def transpose(b):
            src = src_v.at[b]
            dst = dst_v.at[b]

            @functools.partial(plsc.parallel_loop, 0, TCOL // 2, unroll=4)
            def row(r):
                for g in range(8):
                    e = g // 4
                    x = plsc.load_gather(
                        src, [dvecs[g % 4], lanes * 0 + (2 * r + e)]
                    )
                    dst[r, pl.ds(16 * g, 16)] = x

        in_copy(0, 0).start()

        def body(i2, carry):
            for b in range(2):
                i = 2 * i2 + b

                @pl.when(i + 1 < per_w)
                def _():
                    in_copy(i + 1, 1 - b).start()

                in_copy(i, b).wait()

                @pl.when(i >= 2)
                def _():
                    out_copy(i - 2, b).wait()

                transpose(b)
                out_copy(i, b).start()
            return carry

        lax.fori_loop(0, per_w // 2, body, 0)
        out_copy(per_w - 2, 0).wait()
        out_copy(per_w - 1, 1).wait()

        n_tail = (vocab - n_full * TCOL) // 2
        if n_tail:
            @pl.when(wid == NW - 1)
            def _():
                pltpu.sync_copy(tail_hbm, dst_v.at[0].at[pl.ds(0, n_tail)])
                pltpu.sync_copy(
                    dst_v.at[0].at[pl.ds(0, n_tail)],
                    out_hbm.at[pl.ds(n_full * (TCOL // 2), n_tail)],
                )

    return repack_kernel


def _make_gather(n_batch: int, n_seq: int, vocab: int):
    mesh = plsc.VectorSubcoreMesh(core_axis_name="c", subcore_axis_name="s")

    @functools.partial(
        pl.kernel,
        mesh=mesh,
        out_type=jax.ShapeDtypeStruct((n_batch * n_seq * 2, HALF), jnp.float32),
        scratch_types=[
            pltpu.VMEM((n_seq, BLK), jnp.int32),
            pltpu.VMEM((2 * CH,), jnp.int32),
            pltpu.VMEM((2 * CH, HALF), jnp.float32),
            pltpu.SemaphoreType.DMA,
        ],
        compiler_params=pltpu.CompilerParams(
            use_tc_tiling_on_sc=False, needs_layout_passes=False
        ),
    )
    def gather_kernel(tokens_t_hbm, table_hbm, out_hbm, idx_v, idx32_v, rows_v, sem):
        wid = lax.axis_index("s") * NC + lax.axis_index("c")
        base = wid * BLK
        pltpu.sync_copy(tokens_t_hbm.at[:, pl.ds(base, BLK)], idx_v)
        lanes = lax.iota(jnp.int32, 16)

        def body(c, carry):
            j = c // (BLK // CH)
            h = c % (BLK // CH)
            for g in range(CH // 16):
                b = idx_v[j, pl.ds(h * CH + g * 16, 16)] * 2
                plsc.store_scatter(idx32_v, [lanes * 2 + g * 32], b)
                plsc.store_scatter(idx32_v, [lanes * 2 + 1 + g * 32], b + 1)
            pltpu.async_copy(table_hbm.at[idx32_v], rows_v, sem).wait()
            pltpu.sync_copy(
                rows_v,
                out_hbm.at[pl.ds(2 * (j * n_batch + base + h * CH), 2 * CH)],
            )
            return carry

        lax.fori_loop(0, n_seq * (BLK // CH), body, 0)

    return gather_kernel


def kernel(tokens, table):
    n_batch, n_seq = tokens.shape
    vocab = table.shape[0]
    assert n_batch % NW == 0 and n_batch // NW == BLK
    tokens_t = tokens.T.astype(jnp.int32)
    covered = vocab // TCOL * TCOL
    tail128 = table[covered:].reshape(-1, 128)
    table128 = _make_repack(vocab)(table.T, tail128)
    table32 = table128.reshape(2 * vocab, HALF)
    out = _make_gather(n_batch, n_seq, vocab)(tokens_t, table32)
    return out.reshape(n_seq, n_batch, EMBED_DIM).transpose(1, 0, 2)
